# baseline (speedup 1.0000x reference)
import jax
import jax.numpy as jnp
from jax import lax
from jax.experimental import pallas as pl
from jax.experimental.pallas import tpu as pltpu
from jax.experimental.pallas import tpu_sc as plsc

N = 10000
NP = 10240
D_IN = 128
DH = 32
D_OUT = 128
E = 320000
NC, NS, K = 2, 16, 128
NW = NC * NS
RPS = NP // NS
NB = 13
ECH = E // K
CB = ECH // NW
CR = ECH - CB * NW

_f32 = jnp.float32
_mesh = plsc.VectorSubcoreMesh(core_axis_name="c", subcore_axis_name="s")
_sc_params = pltpu.CompilerParams(use_tc_tiling_on_sc=False)


def _slab(wid):
    return CB * wid + jnp.minimum(wid, CR)


def _deg_body(edge_hbm, zeros_hbm, ones_hbm, out_hbm, idx_v, ones_v, acc_sh,
              dsem):
    cid = lax.axis_index("c")
    sid = lax.axis_index("s")
    wid = sid * NC + cid
    row0 = sid * RPS
    bw = _slab(wid)
    cw = CB + (wid < CR).astype(jnp.int32)
    pltpu.sync_copy(zeros_hbm.at[pl.ds(row0, RPS)], acc_sh.at[pl.ds(row0, RPS)])
    pltpu.sync_copy(ones_hbm, ones_v)
    pltpu.sync_copy(edge_hbm.at[1, pl.ds(bw, CB)], idx_v.at[pl.ds(0, CB)])

    @pl.when(wid < CR)
    def _():
        pltpu.sync_copy(edge_hbm.at[1, bw + CB], idx_v.at[CB])

    plsc.subcore_barrier()

    @pl.loop(0, cw)
    def _(c):
        pltpu.async_copy(ones_v, acc_sh.at[idx_v.at[c]], dsem, add=True)

    @pl.loop(0, cw)
    def _(c):
        pltpu.make_async_copy(ones_v, acc_sh.at[idx_v.at[c]], dsem).wait()

    plsc.subcore_barrier()
    pltpu.sync_copy(acc_sh.at[pl.ds(row0, RPS)],
                    out_hbm.at[cid, pl.ds(row0, RPS)])


def _sc_degree(edge_r, zeros16, ones16):
    return pl.kernel(
        _deg_body,
        out_type=jax.ShapeDtypeStruct((NC, NP, 16), _f32),
        mesh=_mesh,
        scratch_types=[
            pltpu.VMEM((CB + 1, K), jnp.int32),
            pltpu.VMEM((K, 16), _f32),
            pltpu.VMEM_SHARED((NP, 16), _f32),
            pltpu.SemaphoreType.DMA,
        ],
        compiler_params=_sc_params,
    )(edge_r, zeros16, ones16)


def _agg_body(y_hbm, edge_hbm, zeros_hbm, out_hbm,
              src_v, dst_v, rows, y_sh, acc_sh, gsem, ssem):
    cid = lax.axis_index("c")
    sid = lax.axis_index("s")
    wid = sid * NC + cid
    row0 = sid * RPS
    bw = _slab(wid)
    pltpu.sync_copy(zeros_hbm.at[pl.ds(row0, RPS)], acc_sh.at[pl.ds(row0, RPS)])
    pltpu.sync_copy(y_hbm.at[pl.ds(row0, RPS)], y_sh.at[pl.ds(row0, RPS)])
    pltpu.sync_copy(edge_hbm.at[0, pl.ds(bw, CB)], src_v.at[pl.ds(0, CB)])
    pltpu.sync_copy(edge_hbm.at[1, pl.ds(bw, CB)], dst_v.at[pl.ds(0, CB)])

    @pl.when(wid < CR)
    def _():
        pltpu.sync_copy(edge_hbm.at[0, bw + CB], src_v.at[CB])
        pltpu.sync_copy(edge_hbm.at[1, bw + CB], dst_v.at[CB])

    plsc.subcore_barrier()

    def gather_start(c, b):
        pltpu.async_copy(y_sh.at[src_v.at[c]], rows.at[b], gsem.at[b])

    def gather_wait(c, b):
        pltpu.make_async_copy(y_sh.at[src_v.at[c]], rows.at[b],
                              gsem.at[b]).wait()

    def scat_start(c, b):
        pltpu.async_copy(rows.at[b], acc_sh.at[dst_v.at[c]], ssem.at[b],
                         add=True)

    def scat_wait(c, b):
        pltpu.make_async_copy(rows.at[b], acc_sh.at[dst_v.at[c]],
                              ssem.at[b]).wait()

    for b in range(NB):
        gather_start(b, b)

    NG = CB // NB

    @pl.loop(0, NG)
    def _(i):
        c0 = i * NB
        for b in range(NB):
            gather_wait(c0 + b, b)
            scat_start(c0 + b, b)
            scat_wait(c0 + b, b)

            @pl.when(i < NG - 1)
            def _():
                gather_start(c0 + NB + b, b)

    @pl.when(wid < CR)
    def _():
        pltpu.sync_copy(y_sh.at[src_v.at[CB]], rows.at[0])
        pltpu.sync_copy(rows.at[0], acc_sh.at[dst_v.at[CB]], add=True)

    plsc.subcore_barrier()
    pltpu.sync_copy(acc_sh.at[pl.ds(row0, RPS)],
                    out_hbm.at[cid, pl.ds(row0, RPS)])


def _sc_aggregate(y, edge_r, zeros32):
    return pl.kernel(
        _agg_body,
        out_type=jax.ShapeDtypeStruct((NC, NP, DH), _f32),
        mesh=_mesh,
        scratch_types=[
            pltpu.VMEM((CB + 1, K), jnp.int32),
            pltpu.VMEM((CB + 1, K), jnp.int32),
            pltpu.VMEM((NB, K, DH), _f32),
            pltpu.VMEM_SHARED((NP, DH), _f32),
            pltpu.VMEM_SHARED((NP, DH), _f32),
            pltpu.SemaphoreType.DMA((NB,)),
            pltpu.SemaphoreType.DMA((NB,)),
        ],
        compiler_params=_sc_params,
    )(y, edge_r, zeros32)


def _mm1_body(x_ref, w_ref, o_ref):
    o_ref[...] = jnp.dot(x_ref[...], w_ref[...], preferred_element_type=_f32)


def _scale_body(degacc_ref, xw_ref, dis_ref, y_ref):
    deg = degacc_ref[0] + degacc_ref[1] + 1.0
    dis = lax.rsqrt(deg)
    dis_ref[...] = dis
    y_ref[...] = xw_ref[...] * dis[:, 0:1]


def _layer1_body(acc_ref, y_ref, dis_ref, b_ref, o_ref):
    dis = dis_ref[:, 0:1]
    agg = dis * (acc_ref[0] + acc_ref[1] + y_ref[...])
    h = jnp.maximum(agg + b_ref[...], 0.0)
    o_ref[...] = dis * h


def _layer2_body(acc_ref, y_ref, dis_ref, w_ref, b_ref, o_ref):
    dis = dis_ref[:, 0:1]
    agg = dis * (acc_ref[0] + acc_ref[1] + y_ref[...])
    z = jnp.dot(agg, w_ref[...], preferred_element_type=_f32) + b_ref[...]
    o_ref[...] = jax.nn.sigmoid(z)


def kernel(x, edge_index, W1, b1, W2, b2):
    edge_r = edge_index.reshape(2, ECH, K)
    x_p = jnp.pad(x, ((0, NP - N), (0, 0)))
    zeros16 = jnp.zeros((NP, 16), _f32)
    ones16 = jnp.ones((K, 16), _f32)
    zeros32 = jnp.zeros((NP, DH), _f32)
    b1r = b1.reshape(1, DH)
    b2r = b2.reshape(1, D_OUT)

    degacc = _sc_degree(edge_r, zeros16, ones16)
    xw = pl.pallas_call(
        _mm1_body,
        out_shape=jax.ShapeDtypeStruct((NP, DH), _f32),
    )(x_p, W1)

    dis, y1 = pl.pallas_call(
        _scale_body,
        out_shape=[jax.ShapeDtypeStruct((NP, 16), _f32),
                   jax.ShapeDtypeStruct((NP, DH), _f32)],
    )(degacc, xw)

    acc1 = _sc_aggregate(y1, edge_r, zeros32)

    y2 = pl.pallas_call(
        _layer1_body,
        out_shape=jax.ShapeDtypeStruct((NP, DH), _f32),
    )(acc1, y1, dis, b1r)

    acc2 = _sc_aggregate(y2, edge_r, zeros32)

    out = pl.pallas_call(
        _layer2_body,
        out_shape=jax.ShapeDtypeStruct((NP, D_OUT), _f32),
    )(acc2, y2, dis, W2, b2r)

    return out[:N]


# speedup vs baseline: 1.0071x; 1.0071x over previous
import jax
import jax.numpy as jnp
from jax import lax
from jax.experimental import pallas as pl
from jax.experimental.pallas import tpu as pltpu
from jax.experimental.pallas import tpu_sc as plsc

N = 10000
NP = 10240
D_IN = 128
DH = 32
D_OUT = 128
E = 320000
NC, NS, K = 2, 16, 128
NW = NC * NS
RPS = NP // NS
NB = 6
ECH = E // K
CB = ECH // NW
CR = ECH - CB * NW

_f32 = jnp.float32
_mesh = plsc.VectorSubcoreMesh(core_axis_name="c", subcore_axis_name="s")
_sc_params = pltpu.CompilerParams(use_tc_tiling_on_sc=False)


def _slab(wid):
    return CB * wid + jnp.minimum(wid, CR)


def _deg_body(edge_hbm, zeros_hbm, ones_hbm, out_hbm, idx_v, ones_v, acc_sh,
              dsem):
    cid = lax.axis_index("c")
    sid = lax.axis_index("s")
    wid = sid * NC + cid
    row0 = sid * RPS
    bw = _slab(wid)
    cw = CB + (wid < CR).astype(jnp.int32)
    pltpu.sync_copy(zeros_hbm.at[pl.ds(row0, RPS)], acc_sh.at[pl.ds(row0, RPS)])
    pltpu.sync_copy(ones_hbm, ones_v)
    pltpu.sync_copy(edge_hbm.at[1, pl.ds(bw, CB)], idx_v.at[pl.ds(0, CB)])

    @pl.when(wid < CR)
    def _():
        pltpu.sync_copy(edge_hbm.at[1, bw + CB], idx_v.at[CB])

    plsc.subcore_barrier()

    @pl.loop(0, cw)
    def _(c):
        pltpu.async_copy(ones_v, acc_sh.at[idx_v.at[c]], dsem, add=True)

    @pl.loop(0, cw)
    def _(c):
        pltpu.make_async_copy(ones_v, acc_sh.at[idx_v.at[c]], dsem).wait()

    plsc.subcore_barrier()
    pltpu.sync_copy(acc_sh.at[pl.ds(row0, RPS)],
                    out_hbm.at[cid, pl.ds(row0, RPS)])


def _sc_degree(edge_r, zeros16, ones16):
    return pl.kernel(
        _deg_body,
        out_type=jax.ShapeDtypeStruct((NC, NP, 16), _f32),
        mesh=_mesh,
        scratch_types=[
            pltpu.VMEM((CB + 1, K), jnp.int32),
            pltpu.VMEM((K, 16), _f32),
            pltpu.VMEM_SHARED((NP, 16), _f32),
            pltpu.SemaphoreType.DMA,
        ],
        compiler_params=_sc_params,
    )(edge_r, zeros16, ones16)


def _agg_body(y_hbm, edge_hbm, zeros_hbm, out_hbm,
              src_v, dst_v, rows, y_sh, acc_sh, gsem, ssem):
    cid = lax.axis_index("c")
    sid = lax.axis_index("s")
    wid = sid * NC + cid
    row0 = sid * RPS
    bw = _slab(wid)
    pltpu.sync_copy(zeros_hbm.at[pl.ds(row0, RPS)], acc_sh.at[pl.ds(row0, RPS)])
    pltpu.sync_copy(y_hbm.at[pl.ds(row0, RPS)], y_sh.at[pl.ds(row0, RPS)])
    pltpu.sync_copy(edge_hbm.at[0, pl.ds(bw, CB)], src_v.at[pl.ds(0, CB)])
    pltpu.sync_copy(edge_hbm.at[1, pl.ds(bw, CB)], dst_v.at[pl.ds(0, CB)])

    @pl.when(wid < CR)
    def _():
        pltpu.sync_copy(edge_hbm.at[0, bw + CB], src_v.at[CB])
        pltpu.sync_copy(edge_hbm.at[1, bw + CB], dst_v.at[CB])

    plsc.subcore_barrier()

    def gather_start(c, b):
        pltpu.async_copy(y_sh.at[src_v.at[c]], rows.at[b], gsem.at[b])

    def gather_wait(c, b):
        pltpu.make_async_copy(y_sh.at[src_v.at[c]], rows.at[b],
                              gsem.at[b]).wait()

    def scat_start(c, b):
        pltpu.async_copy(rows.at[b], acc_sh.at[dst_v.at[c]], ssem.at[b],
                         add=True)

    def scat_wait(c, b):
        pltpu.make_async_copy(rows.at[b], acc_sh.at[dst_v.at[c]],
                              ssem.at[b]).wait()

    for b in range(NB):
        gather_start(b, b)

    NG = CB // NB

    @pl.loop(0, NG)
    def _(i):
        c0 = i * NB
        for b in range(NB):
            gather_wait(c0 + b, b)
            scat_start(c0 + b, b)
            scat_wait(c0 + b, b)

            @pl.when(i < NG - 1)
            def _():
                gather_start(c0 + NB + b, b)

    @pl.when(wid < CR)
    def _():
        pltpu.sync_copy(y_sh.at[src_v.at[CB]], rows.at[0])
        pltpu.sync_copy(rows.at[0], acc_sh.at[dst_v.at[CB]], add=True)

    plsc.subcore_barrier()
    pltpu.sync_copy(acc_sh.at[pl.ds(row0, RPS)],
                    out_hbm.at[cid, pl.ds(row0, RPS)])


def _sc_aggregate(y, edge_r, zeros32):
    return pl.kernel(
        _agg_body,
        out_type=jax.ShapeDtypeStruct((NC, NP, DH), _f32),
        mesh=_mesh,
        scratch_types=[
            pltpu.VMEM((CB + 1, K), jnp.int32),
            pltpu.VMEM((CB + 1, K), jnp.int32),
            pltpu.VMEM((NB, K, DH), _f32),
            pltpu.VMEM_SHARED((NP, DH), _f32),
            pltpu.VMEM_SHARED((NP, DH), _f32),
            pltpu.SemaphoreType.DMA((NB,)),
            pltpu.SemaphoreType.DMA((NB,)),
        ],
        compiler_params=_sc_params,
    )(y, edge_r, zeros32)


def _mm1s_body(degacc_ref, x_ref, w_ref, y_ref, dis_ref):
    deg = degacc_ref[0] + degacc_ref[1] + 1.0
    dis = lax.rsqrt(deg)
    dis_ref[...] = dis
    xs = x_ref[...] * dis[:, 0:1]
    y_ref[...] = jnp.dot(xs, w_ref[...], preferred_element_type=_f32)


def _layer1_body(acc_ref, y_ref, dis_ref, b_ref, o_ref):
    dis = dis_ref[:, 0:1]
    agg = dis * (acc_ref[0] + acc_ref[1] + y_ref[...])
    h = jnp.maximum(agg + b_ref[...], 0.0)
    o_ref[...] = dis * h


def _layer2_body(acc_ref, y_ref, dis_ref, w_ref, b_ref, o_ref):
    dis = dis_ref[:, 0:1]
    agg = dis * (acc_ref[0] + acc_ref[1] + y_ref[...])
    z = jnp.dot(agg, w_ref[...], preferred_element_type=_f32) + b_ref[...]
    o_ref[...] = jax.nn.sigmoid(z)


def kernel(x, edge_index, W1, b1, W2, b2):
    edge_r = edge_index.reshape(2, ECH, K)
    x_p = jnp.pad(x, ((0, NP - N), (0, 0)))
    zeros16 = jnp.zeros((NP, 16), _f32)
    ones16 = jnp.ones((K, 16), _f32)
    zeros32 = jnp.zeros((NP, DH), _f32)
    b1r = b1.reshape(1, DH)
    b2r = b2.reshape(1, D_OUT)

    degacc = _sc_degree(edge_r, zeros16, ones16)
    y1, dis = pl.pallas_call(
        _mm1s_body,
        out_shape=[jax.ShapeDtypeStruct((NP, DH), _f32),
                   jax.ShapeDtypeStruct((NP, 16), _f32)],
    )(degacc, x_p, W1)

    acc1 = _sc_aggregate(y1, edge_r, zeros32)

    y2 = pl.pallas_call(
        _layer1_body,
        out_shape=jax.ShapeDtypeStruct((NP, DH), _f32),
    )(acc1, y1, dis, b1r)

    acc2 = _sc_aggregate(y2, edge_r, zeros32)

    out = pl.pallas_call(
        _layer2_body,
        out_shape=jax.ShapeDtypeStruct((NP, D_OUT), _f32),
    )(acc2, y2, dis, W2, b2r)

    return out[:N]
